# baseline (speedup 1.0000x reference)
import functools

import jax
import jax.numpy as jnp
from jax import lax
from jax.experimental import pallas as pl
from jax.experimental.pallas import tpu as pltpu
from jax.experimental.pallas import tpu_sc as plsc

N = 10000
E = 320000
DF = 128
DH = 128
DC = 64

NC = 2
NS = 16
NW = NC * NS
N_PAD = 10240
ROWS_PER_TILE = N_PAD // NS
E_PER_TILE = E // NW
CW = 128
CHUNKS = 80
E_TILE_PAD = CHUNKS * CW
PHASES = 2
PCH = CHUNKS // PHASES
NBUF = 2

_N_SCORES = DF * DH + DH * DH + DH * DC
_K_ORDER = 1 + int(round(0.01 * 50.0 * (_N_SCORES - 1)))


def _monokeys(s):
    u = lax.bitcast_convert_type(s, jnp.uint32)
    neg = (u >> jnp.uint32(31)) == jnp.uint32(1)
    return jnp.where(neg, ~u, u | jnp.uint32(0x80000000))


def _prep_body(s0, s1, s2, w0, w1, w2, o0, o1, o2):
    k0 = _monokeys(s0[...])
    k1 = _monokeys(s1[...])
    k2 = _monokeys(s2[...])
    K = jnp.uint32(0)
    for b in range(31, -1, -1):
        trial = K | jnp.uint32(1 << b)
        cnt = (jnp.sum((k0 < trial).astype(jnp.int32))
               + jnp.sum((k1 < trial).astype(jnp.int32))
               + jnp.sum((k2 < trial).astype(jnp.int32)))
        K = jnp.where(cnt >= _K_ORDER, K, trial)
    top = (K >> jnp.uint32(31)) == jnp.uint32(1)
    u = jnp.where(top, K ^ jnp.uint32(0x80000000), ~K)
    thr = lax.bitcast_convert_type(u, jnp.float32)
    o0[...] = w0[...] * (s0[...] >= thr).astype(jnp.float32)
    o1[...] = w1[...] * (s1[...] >= thr).astype(jnp.float32)
    w2p = jnp.pad(w2[...] * (s2[...] >= thr).astype(jnp.float32),
                  ((0, 0), (0, DH - DC)))
    o2[...] = w2p


_prep_call = pl.pallas_call(
    _prep_body,
    out_shape=[
        jax.ShapeDtypeStruct((DF, DH), jnp.float32),
        jax.ShapeDtypeStruct((DH, DH), jnp.float32),
        jax.ShapeDtypeStruct((DH, DH), jnp.float32),
    ],
)

_BR = 1024
_GRID = N_PAD // _BR


def _layer0_body(x, w, dga, dgb, o):
    dinv = lax.rsqrt(dga[0, :, :1] + dgb[0, :, :1] + 1.0)
    h = jnp.dot(x[...], w[...], preferred_element_type=jnp.float32)
    o[...] = h * dinv


def _layer_body(a, b, hp, w, dga, dgb, o):
    dinv = lax.rsqrt(dga[0, :, :1] + dgb[0, :, :1] + 1.0)
    agg = a[0] + b[0] - hp[...]
    act = jnp.maximum(agg * dinv, 0.0)
    o[...] = jnp.dot(act, w[...], preferred_element_type=jnp.float32) * dinv


def _final_body(a, b, hp, dga, dgb, o):
    dinv = lax.rsqrt(dga[0, :, :1] + dgb[0, :, :1] + 1.0)
    agg = a[0] + b[0] - hp[...]
    o[...] = (agg * dinv)[:, :DC]


def _row_spec(d):
    return pl.BlockSpec((_BR, d), lambda i: (i, 0))


def _stk_spec(half, d):
    return pl.BlockSpec((1, _BR, d), lambda i, _h=half: (_h, i, 0))


def _full_spec(r, c):
    return pl.BlockSpec((r, c), lambda i: (0, 0))


def _make_layer0():
    return pl.pallas_call(
        _layer0_body,
        grid=(_GRID,),
        in_specs=[_row_spec(DF), _full_spec(DF, DH),
                  _stk_spec(0, DH), _stk_spec(1, DH)],
        out_specs=_row_spec(DH),
        out_shape=jax.ShapeDtypeStruct((N_PAD, DH), jnp.float32),
    )


def _make_layer():
    return pl.pallas_call(
        _layer_body,
        grid=(_GRID,),
        in_specs=[_stk_spec(0, DH), _stk_spec(1, DH), _row_spec(DH),
                  _full_spec(DH, DH), _stk_spec(0, DH), _stk_spec(1, DH)],
        out_specs=_row_spec(DH),
        out_shape=jax.ShapeDtypeStruct((N_PAD, DH), jnp.float32),
    )


def _make_final():
    return pl.pallas_call(
        _final_body,
        grid=(_GRID,),
        in_specs=[_stk_spec(0, DH), _stk_spec(1, DH), _row_spec(DH),
                  _stk_spec(0, DH), _stk_spec(1, DH)],
        out_specs=_row_spec(DC),
        out_shape=jax.ShapeDtypeStruct((N_PAD, DC), jnp.float32),
    )


@functools.cache
def _mesh():
    return plsc.VectorSubcoreMesh(core_axis_name="c", subcore_axis_name="s",
                                  num_cores=NC, num_subcores=NS)


def _deg_kernel_body(dst_hbm, ones_hbm, zeros_hbm, out_hbm,
                     dst_v, ones_v, acc_sh):
    c = lax.axis_index("c")
    s = lax.axis_index("s")
    wid = s * NC + c
    lo = s * ROWS_PER_TILE
    pltpu.sync_copy(dst_hbm.at[wid], dst_v)
    pltpu.sync_copy(ones_hbm, ones_v)
    pltpu.sync_copy(zeros_hbm.at[pl.ds(lo, ROWS_PER_TILE)],
                    acc_sh.at[pl.ds(lo, ROWS_PER_TILE)])
    plsc.subcore_barrier()

    def body(j, carry):
        pltpu.sync_copy(ones_v, acc_sh.at[dst_v.at[j]], add=True)
        return carry

    lax.fori_loop(0, CHUNKS, body, 0)
    plsc.subcore_barrier()
    pltpu.sync_copy(acc_sh.at[pl.ds(lo, ROWS_PER_TILE)],
                    out_hbm.at[c, pl.ds(lo, ROWS_PER_TILE)])


@functools.cache
def _make_deg():
    return pl.kernel(
        _deg_kernel_body,
        out_type=jax.ShapeDtypeStruct((NC, N_PAD, DH), jnp.float32),
        mesh=_mesh(),
        scratch_types=[
            pltpu.VMEM((CHUNKS, CW), jnp.int32),
            pltpu.VMEM((CW, DH), jnp.float32),
            pltpu.VMEM_SHARED((N_PAD, DH), jnp.float32),
        ],
        compiler_params=pltpu.CompilerParams(use_tc_tiling_on_sc=False),
    )


def _agg_kernel_body(h_hbm, src_hbm, dst_hbm, out_hbm,
                     src_v, dst_v, rows_v, acc_sh):
    c = lax.axis_index("c")
    s = lax.axis_index("s")
    wid = s * NC + c
    lo = s * ROWS_PER_TILE
    pltpu.sync_copy(src_hbm.at[wid], src_v)
    pltpu.sync_copy(dst_hbm.at[wid], dst_v)
    pltpu.sync_copy(h_hbm.at[pl.ds(lo, ROWS_PER_TILE)],
                    acc_sh.at[pl.ds(lo, ROWS_PER_TILE)])
    plsc.subcore_barrier()

    def body(j, carry):
        pltpu.sync_copy(h_hbm.at[src_v.at[j]], rows_v)
        pltpu.sync_copy(rows_v, acc_sh.at[dst_v.at[j]], add=True)
        return carry

    lax.fori_loop(0, CHUNKS, body, 0)
    plsc.subcore_barrier()
    pltpu.sync_copy(acc_sh.at[pl.ds(lo, ROWS_PER_TILE)],
                    out_hbm.at[c, pl.ds(lo, ROWS_PER_TILE)])


@functools.cache
def _make_agg():
    return pl.kernel(
        _agg_kernel_body,
        out_type=jax.ShapeDtypeStruct((NC, N_PAD, DH), jnp.float32),
        mesh=_mesh(),
        scratch_types=[
            pltpu.VMEM((CHUNKS, CW), jnp.int32),
            pltpu.VMEM((CHUNKS, CW), jnp.int32),
            pltpu.VMEM((CW, DH), jnp.float32),
            pltpu.VMEM_SHARED((N_PAD, DH), jnp.float32),
        ],
        compiler_params=pltpu.CompilerParams(use_tc_tiling_on_sc=False),
    )


def kernel(x, edge_index, W0, S0, W1, S1, W2, S2):
    src = edge_index[0].astype(jnp.int32).reshape(NW, E_PER_TILE)
    dst = edge_index[1].astype(jnp.int32).reshape(NW, E_PER_TILE)
    src = jnp.pad(src, ((0, 0), (0, E_TILE_PAD - E_PER_TILE)),
                  constant_values=0).reshape(NW, CHUNKS, CW)
    dst = jnp.pad(dst, ((0, 0), (0, E_TILE_PAD - E_PER_TILE)),
                  constant_values=N).reshape(NW, CHUNKS, CW)

    x_pad = jnp.pad(x, ((0, N_PAD - N), (0, 0)))
    zeros128 = jnp.zeros((N_PAD, DH), jnp.float32)
    ones128 = jnp.ones((CW, DH), jnp.float32)

    Wm0, Wm1, Wm2p = _prep_call(S0, S1, S2, W0, W1, W2)
    dg = _make_deg()(dst, ones128, zeros128)

    h = _make_layer0()(x_pad, Wm0, dg, dg)
    a = _make_agg()(h, src, dst)
    h = _make_layer()(a, a, h, Wm1, dg, dg)
    a = _make_agg()(h, src, dst)
    h = _make_layer()(a, a, h, Wm2p, dg, dg)
    a = _make_agg()(h, src, dst)
    out = _make_final()(a, a, h, dg, dg)
    return out[:N]


# speedup vs baseline: 1.0018x; 1.0018x over previous
import functools

import jax
import jax.numpy as jnp
from jax import lax
from jax.experimental import pallas as pl
from jax.experimental.pallas import tpu as pltpu
from jax.experimental.pallas import tpu_sc as plsc

N = 10000
E = 320000
DF = 128
DH = 128
DC = 64

NC = 2
NS = 16
NW = NC * NS
N_PAD = 10240
ROWS_PER_TILE = N_PAD // NS
E_PER_TILE = E // NW
CW = 128
CHUNKS = 80
E_TILE_PAD = CHUNKS * CW
PHASES = 2
PCH = CHUNKS // PHASES
NBUF = 2

_N_SCORES = DF * DH + DH * DH + DH * DC
_K_ORDER = 1 + int(round(0.01 * 50.0 * (_N_SCORES - 1)))


def _monokeys(s):
    u = lax.bitcast_convert_type(s, jnp.uint32)
    neg = (u >> jnp.uint32(31)) == jnp.uint32(1)
    return jnp.where(neg, ~u, u | jnp.uint32(0x80000000))


def _prep_body(s0, s1, s2, w0, w1, w2, o0, o1, o2):
    k0 = _monokeys(s0[...])
    k1 = _monokeys(s1[...])
    k2 = _monokeys(s2[...])
    K = jnp.uint32(0)
    for b in range(31, -1, -1):
        trial = K | jnp.uint32(1 << b)
        cnt = (jnp.sum((k0 < trial).astype(jnp.int32))
               + jnp.sum((k1 < trial).astype(jnp.int32))
               + jnp.sum((k2 < trial).astype(jnp.int32)))
        K = jnp.where(cnt >= _K_ORDER, K, trial)
    top = (K >> jnp.uint32(31)) == jnp.uint32(1)
    u = jnp.where(top, K ^ jnp.uint32(0x80000000), ~K)
    thr = lax.bitcast_convert_type(u, jnp.float32)
    o0[...] = w0[...] * (s0[...] >= thr).astype(jnp.float32)
    o1[...] = w1[...] * (s1[...] >= thr).astype(jnp.float32)
    w2p = jnp.pad(w2[...] * (s2[...] >= thr).astype(jnp.float32),
                  ((0, 0), (0, DH - DC)))
    o2[...] = w2p


_prep_call = pl.pallas_call(
    _prep_body,
    out_shape=[
        jax.ShapeDtypeStruct((DF, DH), jnp.float32),
        jax.ShapeDtypeStruct((DH, DH), jnp.float32),
        jax.ShapeDtypeStruct((DH, DH), jnp.float32),
    ],
)

_BR = 1024
_GRID = N_PAD // _BR


def _layer0_body(x, w, dga, dgb, o):
    dinv = lax.rsqrt(dga[0, :, :1] + dgb[0, :, :1] + 1.0)
    h = jnp.dot(x[...], w[...], preferred_element_type=jnp.float32)
    o[...] = h * dinv


def _layer_body(a, b, hp, w, dga, dgb, o):
    dinv = lax.rsqrt(dga[0, :, :1] + dgb[0, :, :1] + 1.0)
    agg = a[0] + b[0] - hp[...]
    act = jnp.maximum(agg * dinv, 0.0)
    o[...] = jnp.dot(act, w[...], preferred_element_type=jnp.float32) * dinv


def _final_body(a, b, hp, dga, dgb, o):
    dinv = lax.rsqrt(dga[0, :, :1] + dgb[0, :, :1] + 1.0)
    agg = a[0] + b[0] - hp[...]
    o[...] = (agg * dinv)[:, :DC]


def _row_spec(d):
    return pl.BlockSpec((_BR, d), lambda i: (i, 0))


def _stk_spec(half, d):
    return pl.BlockSpec((1, _BR, d), lambda i, _h=half: (_h, i, 0))


def _full_spec(r, c):
    return pl.BlockSpec((r, c), lambda i: (0, 0))


def _make_layer0():
    return pl.pallas_call(
        _layer0_body,
        grid=(_GRID,),
        in_specs=[_row_spec(DF), _full_spec(DF, DH),
                  _stk_spec(0, DH), _stk_spec(1, DH)],
        out_specs=_row_spec(DH),
        out_shape=jax.ShapeDtypeStruct((N_PAD, DH), jnp.float32),
    )


def _make_layer():
    return pl.pallas_call(
        _layer_body,
        grid=(_GRID,),
        in_specs=[_stk_spec(0, DH), _stk_spec(1, DH), _row_spec(DH),
                  _full_spec(DH, DH), _stk_spec(0, DH), _stk_spec(1, DH)],
        out_specs=_row_spec(DH),
        out_shape=jax.ShapeDtypeStruct((N_PAD, DH), jnp.float32),
    )


def _make_final():
    return pl.pallas_call(
        _final_body,
        grid=(_GRID,),
        in_specs=[_stk_spec(0, DH), _stk_spec(1, DH), _row_spec(DH),
                  _stk_spec(0, DH), _stk_spec(1, DH)],
        out_specs=_row_spec(DC),
        out_shape=jax.ShapeDtypeStruct((N_PAD, DC), jnp.float32),
    )


@functools.cache
def _mesh():
    return plsc.VectorSubcoreMesh(core_axis_name="c", subcore_axis_name="s",
                                  num_cores=NC, num_subcores=NS)


def _deg_kernel_body(dst_hbm, ones_hbm, zeros_hbm, out_hbm,
                     dst_v, ones_v, acc_sh):
    c = lax.axis_index("c")
    s = lax.axis_index("s")
    wid = s * NC + c
    lo = s * ROWS_PER_TILE
    pltpu.sync_copy(dst_hbm.at[wid], dst_v)
    pltpu.sync_copy(ones_hbm, ones_v)
    pltpu.sync_copy(zeros_hbm.at[pl.ds(lo, ROWS_PER_TILE)],
                    acc_sh.at[pl.ds(lo, ROWS_PER_TILE)])
    plsc.subcore_barrier()

    def body(j, carry):
        pltpu.sync_copy(ones_v, acc_sh.at[dst_v.at[j]], add=True)
        return carry

    lax.fori_loop(0, CHUNKS, body, 0)
    plsc.subcore_barrier()
    pltpu.sync_copy(acc_sh.at[pl.ds(lo, ROWS_PER_TILE)],
                    out_hbm.at[c, pl.ds(lo, ROWS_PER_TILE)])


@functools.cache
def _make_deg():
    return pl.kernel(
        _deg_kernel_body,
        out_type=jax.ShapeDtypeStruct((NC, N_PAD, DH), jnp.float32),
        mesh=_mesh(),
        scratch_types=[
            pltpu.VMEM((CHUNKS, CW), jnp.int32),
            pltpu.VMEM((CW, DH), jnp.float32),
            pltpu.VMEM_SHARED((N_PAD, DH), jnp.float32),
        ],
        compiler_params=pltpu.CompilerParams(use_tc_tiling_on_sc=False),
    )


def _agg_kernel_body(h_hbm, src_hbm, dst_hbm, out_hbm,
                     src_v, dst_v, rows_v, acc_sh):
    c = lax.axis_index("c")
    s = lax.axis_index("s")
    wid = s * NC + c
    lo = s * ROWS_PER_TILE
    pltpu.sync_copy(src_hbm.at[wid], src_v)
    pltpu.sync_copy(dst_hbm.at[wid], dst_v)
    pltpu.sync_copy(h_hbm.at[pl.ds(lo, ROWS_PER_TILE)],
                    acc_sh.at[pl.ds(lo, ROWS_PER_TILE)])
    plsc.subcore_barrier()

    def body(j, carry):
        pltpu.sync_copy(h_hbm.at[src_v.at[j]], rows_v)
        pltpu.sync_copy(rows_v, acc_sh.at[dst_v.at[j]], add=True)
        return carry

    lax.fori_loop(0, CHUNKS, body, 0)
    plsc.subcore_barrier()
    pltpu.sync_copy(acc_sh.at[pl.ds(lo, ROWS_PER_TILE)],
                    out_hbm.at[c, pl.ds(lo, ROWS_PER_TILE)])


@functools.cache
def _make_agg():
    return pl.kernel(
        _agg_kernel_body,
        out_type=jax.ShapeDtypeStruct((NC, N_PAD, DH), jnp.float32),
        mesh=_mesh(),
        scratch_types=[
            pltpu.VMEM((CHUNKS, CW), jnp.int32),
            pltpu.VMEM((CHUNKS, CW), jnp.int32),
            pltpu.VMEM((CW, DH), jnp.float32),
            pltpu.VMEM_SHARED((N_PAD, DH), jnp.float32),
        ],
        compiler_params=pltpu.CompilerParams(use_tc_tiling_on_sc=False),
    )


def kernel(x, edge_index, W0, S0, W1, S1, W2, S2):
    src = edge_index[0].astype(jnp.int32).reshape(NW, E_PER_TILE)
    dst = edge_index[1].astype(jnp.int32).reshape(NW, E_PER_TILE)
    pad_n = E_TILE_PAD - E_PER_TILE
    pad_dst = N + (jnp.arange(pad_n, dtype=jnp.int32) % (N_PAD - N))
    src = jnp.pad(src, ((0, 0), (0, pad_n)),
                  constant_values=0).reshape(NW, CHUNKS, CW)
    dst = jnp.concatenate(
        [dst, jnp.broadcast_to(pad_dst, (NW, pad_n))],
        axis=1).reshape(NW, CHUNKS, CW)

    x_pad = jnp.pad(x, ((0, N_PAD - N), (0, 0)))
    zeros128 = jnp.zeros((N_PAD, DH), jnp.float32)
    ones128 = jnp.ones((CW, DH), jnp.float32)

    Wm0, Wm1, Wm2p = _prep_call(S0, S1, S2, W0, W1, W2)
    dg = _make_deg()(dst, ones128, zeros128)

    h = _make_layer0()(x_pad, Wm0, dg, dg)
    a = _make_agg()(h, src, dst)
    h = _make_layer()(a, a, h, Wm1, dg, dg)
    a = _make_agg()(h, src, dst)
    h = _make_layer()(a, a, h, Wm2p, dg, dg)
    a = _make_agg()(h, src, dst)
    out = _make_final()(a, a, h, dg, dg)
    return out[:N]


# speedup vs baseline: 1.4349x; 1.4324x over previous
import functools

import jax
import jax.numpy as jnp
from jax import lax
from jax.experimental import pallas as pl
from jax.experimental.pallas import tpu as pltpu
from jax.experimental.pallas import tpu_sc as plsc

N = 10000
E = 320000
DF = 128
DH = 128
DC = 64

NC = 2
NS = 16
NW = NC * NS
N_PAD = 10240
ROWS_PER_TILE = N_PAD // NS
E_PER_TILE = E // NW
CW = 128
CHUNKS = 79
E_TILE_PAD = CHUNKS * CW
PHASES = 2
PCH = CHUNKS // PHASES
NBUF = 2

_N_SCORES = DF * DH + DH * DH + DH * DC
_K_ORDER = 1 + int(round(0.01 * 50.0 * (_N_SCORES - 1)))


def _monokeys(s):
    u = lax.bitcast_convert_type(s, jnp.uint32)
    neg = (u >> jnp.uint32(31)) == jnp.uint32(1)
    return jnp.where(neg, ~u, u | jnp.uint32(0x80000000))


def _prep_body(s0, s1, s2, w0, w1, w2, o0, o1, o2):
    k0 = _monokeys(s0[...])
    k1 = _monokeys(s1[...])
    k2 = _monokeys(s2[...])
    K = jnp.uint32(0)
    for b in range(31, -1, -1):
        trial = K | jnp.uint32(1 << b)
        cnt = (jnp.sum((k0 < trial).astype(jnp.int32))
               + jnp.sum((k1 < trial).astype(jnp.int32))
               + jnp.sum((k2 < trial).astype(jnp.int32)))
        K = jnp.where(cnt >= _K_ORDER, K, trial)
    top = (K >> jnp.uint32(31)) == jnp.uint32(1)
    u = jnp.where(top, K ^ jnp.uint32(0x80000000), ~K)
    thr = lax.bitcast_convert_type(u, jnp.float32)
    o0[...] = w0[...] * (s0[...] >= thr).astype(jnp.float32)
    o1[...] = w1[...] * (s1[...] >= thr).astype(jnp.float32)
    w2p = jnp.pad(w2[...] * (s2[...] >= thr).astype(jnp.float32),
                  ((0, 0), (0, DH - DC)))
    o2[...] = w2p


_prep_call = pl.pallas_call(
    _prep_body,
    out_shape=[
        jax.ShapeDtypeStruct((DF, DH), jnp.float32),
        jax.ShapeDtypeStruct((DH, DH), jnp.float32),
        jax.ShapeDtypeStruct((DH, DH), jnp.float32),
    ],
)

_BR = 1024
_GRID = N_PAD // _BR


def _layer0_body(x, w, dga, dgb, o):
    dinv = lax.rsqrt(dga[0, :, :1] + dgb[0, :, :1] + 1.0)
    h = jnp.dot(x[...], w[...], preferred_element_type=jnp.float32)
    o[...] = h * dinv


def _layer_body(a, b, hp, w, dga, dgb, o):
    dinv = lax.rsqrt(dga[0, :, :1] + dgb[0, :, :1] + 1.0)
    agg = a[0] + b[0] - hp[...]
    act = jnp.maximum(agg * dinv, 0.0)
    o[...] = jnp.dot(act, w[...], preferred_element_type=jnp.float32) * dinv


def _final_body(a, b, hp, dga, dgb, o):
    dinv = lax.rsqrt(dga[0, :, :1] + dgb[0, :, :1] + 1.0)
    agg = a[0] + b[0] - hp[...]
    o[...] = (agg * dinv)[:, :DC]


def _row_spec(d):
    return pl.BlockSpec((_BR, d), lambda i: (i, 0))


def _stk_spec(half, d):
    return pl.BlockSpec((1, _BR, d), lambda i, _h=half: (_h, i, 0))


def _full_spec(r, c):
    return pl.BlockSpec((r, c), lambda i: (0, 0))


def _make_layer0():
    return pl.pallas_call(
        _layer0_body,
        grid=(_GRID,),
        in_specs=[_row_spec(DF), _full_spec(DF, DH),
                  _stk_spec(0, DH), _stk_spec(1, DH)],
        out_specs=_row_spec(DH),
        out_shape=jax.ShapeDtypeStruct((N_PAD, DH), jnp.float32),
    )


def _make_layer():
    return pl.pallas_call(
        _layer_body,
        grid=(_GRID,),
        in_specs=[_stk_spec(0, DH), _stk_spec(1, DH), _row_spec(DH),
                  _full_spec(DH, DH), _stk_spec(0, DH), _stk_spec(1, DH)],
        out_specs=_row_spec(DH),
        out_shape=jax.ShapeDtypeStruct((N_PAD, DH), jnp.float32),
    )


def _make_final():
    return pl.pallas_call(
        _final_body,
        grid=(_GRID,),
        in_specs=[_stk_spec(0, DH), _stk_spec(1, DH), _row_spec(DH),
                  _stk_spec(0, DH), _stk_spec(1, DH)],
        out_specs=_row_spec(DC),
        out_shape=jax.ShapeDtypeStruct((N_PAD, DC), jnp.float32),
    )


@functools.cache
def _mesh():
    return plsc.VectorSubcoreMesh(core_axis_name="c", subcore_axis_name="s",
                                  num_cores=NC, num_subcores=NS)


def _deg_kernel_body(dst_hbm, ones_hbm, zeros_hbm, out_hbm,
                     dst_v, ones_v, acc_sh):
    c = lax.axis_index("c")
    s = lax.axis_index("s")
    wid = s * NC + c
    lo = s * ROWS_PER_TILE
    pltpu.sync_copy(dst_hbm.at[wid], dst_v)
    pltpu.sync_copy(ones_hbm, ones_v)
    pltpu.sync_copy(zeros_hbm.at[pl.ds(lo, ROWS_PER_TILE)],
                    acc_sh.at[pl.ds(lo, ROWS_PER_TILE)])
    plsc.subcore_barrier()

    def body(j, carry):
        pltpu.sync_copy(ones_v, acc_sh.at[dst_v.at[j]], add=True)
        return carry

    lax.fori_loop(0, CHUNKS, body, 0)
    plsc.subcore_barrier()
    pltpu.sync_copy(acc_sh.at[pl.ds(lo, ROWS_PER_TILE)],
                    out_hbm.at[c, pl.ds(lo, ROWS_PER_TILE)])


@functools.cache
def _make_deg():
    return pl.kernel(
        _deg_kernel_body,
        out_type=jax.ShapeDtypeStruct((NC, N_PAD, DH), jnp.float32),
        mesh=_mesh(),
        scratch_types=[
            pltpu.VMEM((CHUNKS, CW), jnp.int32),
            pltpu.VMEM((CW, DH), jnp.float32),
            pltpu.VMEM_SHARED((N_PAD, DH), jnp.float32),
        ],
        compiler_params=pltpu.CompilerParams(use_tc_tiling_on_sc=False),
    )


def _agg_kernel_body(h_hbm, src_hbm, dst_hbm, out_hbm,
                     src_v, dst_v, rows_v, acc_sh):
    c = lax.axis_index("c")
    s = lax.axis_index("s")
    wid = s * NC + c
    lo = s * ROWS_PER_TILE
    pltpu.sync_copy(src_hbm.at[wid], src_v)
    pltpu.sync_copy(dst_hbm.at[wid], dst_v)
    pltpu.sync_copy(h_hbm.at[pl.ds(lo, ROWS_PER_TILE)],
                    acc_sh.at[pl.ds(lo, ROWS_PER_TILE)])
    plsc.subcore_barrier()

    def body(j, carry):
        pltpu.sync_copy(h_hbm.at[src_v.at[j]], rows_v)
        pltpu.sync_copy(rows_v, acc_sh.at[dst_v.at[j]], add=True)
        return carry

    lax.fori_loop(0, CHUNKS, body, 0)
    plsc.subcore_barrier()
    pltpu.sync_copy(acc_sh.at[pl.ds(lo, ROWS_PER_TILE)],
                    out_hbm.at[c, pl.ds(lo, ROWS_PER_TILE)])


@functools.cache
def _make_agg():
    return pl.kernel(
        _agg_kernel_body,
        out_type=jax.ShapeDtypeStruct((NC, N_PAD, DH), jnp.float32),
        mesh=_mesh(),
        scratch_types=[
            pltpu.VMEM((CHUNKS, CW), jnp.int32),
            pltpu.VMEM((CHUNKS, CW), jnp.int32),
            pltpu.VMEM((CW, DH), jnp.float32),
            pltpu.VMEM_SHARED((N_PAD, DH), jnp.float32),
        ],
        compiler_params=pltpu.CompilerParams(use_tc_tiling_on_sc=False),
    )


def kernel(x, edge_index, W0, S0, W1, S1, W2, S2):
    src = edge_index[0].astype(jnp.int32).reshape(NW, E_PER_TILE)
    dst = edge_index[1].astype(jnp.int32).reshape(NW, E_PER_TILE)
    pad_n = E_TILE_PAD - E_PER_TILE
    pad_dst = N + (jnp.arange(pad_n, dtype=jnp.int32) % (N_PAD - N))
    src = jnp.pad(src, ((0, 0), (0, pad_n)),
                  constant_values=0).reshape(NW, CHUNKS, CW)
    dst = jnp.concatenate(
        [dst, jnp.broadcast_to(pad_dst, (NW, pad_n))],
        axis=1).reshape(NW, CHUNKS, CW)

    x_pad = jnp.pad(x, ((0, N_PAD - N), (0, 0)))
    zeros128 = jnp.zeros((N_PAD, DH), jnp.float32)
    ones128 = jnp.ones((CW, DH), jnp.float32)

    Wm0, Wm1, Wm2p = _prep_call(S0, S1, S2, W0, W1, W2)
    dg = _make_deg()(dst, ones128, zeros128)

    h = _make_layer0()(x_pad, Wm0, dg, dg)
    a = _make_agg()(h, src, dst)
    h = _make_layer()(a, a, h, Wm1, dg, dg)
    a = _make_agg()(h, src, dst)
    h = _make_layer()(a, a, h, Wm2p, dg, dg)
    a = _make_agg()(h, src, dst)
    out = _make_final()(a, a, h, dg, dg)
    return out[:N]


# speedup vs baseline: 2.2185x; 1.5461x over previous
import functools

import jax
import jax.numpy as jnp
from jax import lax
from jax.experimental import pallas as pl
from jax.experimental.pallas import tpu as pltpu
from jax.experimental.pallas import tpu_sc as plsc

N = 10000
E = 320000
DF = 128
DH = 128
DC = 64

NC = 2
NS = 16
NW = NC * NS
N_PAD = 10240
ROWS_PER_TILE = N_PAD // NS
E_PER_TILE = E // NW
CW = 128
CHUNKS = 79
E_TILE_PAD = CHUNKS * CW
PHASES = 2
PCH = CHUNKS // PHASES
NBUF = 2

_N_SCORES = DF * DH + DH * DH + DH * DC
_K_ORDER = 1 + int(round(0.01 * 50.0 * (_N_SCORES - 1)))


def _monokeys(s):
    u = lax.bitcast_convert_type(s, jnp.uint32)
    neg = (u >> jnp.uint32(31)) == jnp.uint32(1)
    return jnp.where(neg, ~u, u | jnp.uint32(0x80000000))


def _prep_body(s0, s1, s2, w0, w1, w2, o0, o1, o2):
    k0 = _monokeys(s0[...])
    k1 = _monokeys(s1[...])
    k2 = _monokeys(s2[...])
    K = jnp.uint32(0)
    for b in range(31, -1, -1):
        trial = K | jnp.uint32(1 << b)
        cnt = (jnp.sum((k0 < trial).astype(jnp.int32))
               + jnp.sum((k1 < trial).astype(jnp.int32))
               + jnp.sum((k2 < trial).astype(jnp.int32)))
        K = jnp.where(cnt >= _K_ORDER, K, trial)
    top = (K >> jnp.uint32(31)) == jnp.uint32(1)
    u = jnp.where(top, K ^ jnp.uint32(0x80000000), ~K)
    thr = lax.bitcast_convert_type(u, jnp.float32)
    o0[...] = w0[...] * (s0[...] >= thr).astype(jnp.float32)
    o1[...] = w1[...] * (s1[...] >= thr).astype(jnp.float32)
    w2p = jnp.pad(w2[...] * (s2[...] >= thr).astype(jnp.float32),
                  ((0, 0), (0, DH - DC)))
    o2[...] = w2p


_prep_call = pl.pallas_call(
    _prep_body,
    out_shape=[
        jax.ShapeDtypeStruct((DF, DH), jnp.float32),
        jax.ShapeDtypeStruct((DH, DH), jnp.float32),
        jax.ShapeDtypeStruct((DH, DH), jnp.float32),
    ],
)

_BR = 1024
_GRID = N_PAD // _BR


def _layer0_body(x, w, dga, dgb, o):
    dinv = lax.rsqrt(dga[0, :, :1] + dgb[0, :, :1] + 1.0)
    h = jnp.dot(x[...], w[...], preferred_element_type=jnp.float32)
    o[...] = h * dinv


def _layer_body(a, b, hp, w, dga, dgb, o):
    dinv = lax.rsqrt(dga[0, :, :1] + dgb[0, :, :1] + 1.0)
    agg = a[0] + b[0] - hp[...]
    act = jnp.maximum(agg * dinv, 0.0)
    o[...] = jnp.dot(act, w[...], preferred_element_type=jnp.float32) * dinv


def _final_body(a, b, hp, dga, dgb, o):
    dinv = lax.rsqrt(dga[0, :, :1] + dgb[0, :, :1] + 1.0)
    agg = a[0] + b[0] - hp[...]
    o[...] = (agg * dinv)[:, :DC]


def _row_spec(d):
    return pl.BlockSpec((_BR, d), lambda i: (i, 0))


def _stk_spec(half, d):
    return pl.BlockSpec((1, _BR, d), lambda i, _h=half: (_h, i, 0))


def _full_spec(r, c):
    return pl.BlockSpec((r, c), lambda i: (0, 0))


def _make_layer0():
    return pl.pallas_call(
        _layer0_body,
        grid=(_GRID,),
        in_specs=[_row_spec(DF), _full_spec(DF, DH),
                  _stk_spec(0, DH), _stk_spec(1, DH)],
        out_specs=_row_spec(DH),
        out_shape=jax.ShapeDtypeStruct((N_PAD, DH), jnp.float32),
    )


def _make_layer():
    return pl.pallas_call(
        _layer_body,
        grid=(_GRID,),
        in_specs=[_stk_spec(0, DH), _stk_spec(1, DH), _row_spec(DH),
                  _full_spec(DH, DH), _stk_spec(0, DH), _stk_spec(1, DH)],
        out_specs=_row_spec(DH),
        out_shape=jax.ShapeDtypeStruct((N_PAD, DH), jnp.float32),
    )


def _make_final():
    return pl.pallas_call(
        _final_body,
        grid=(_GRID,),
        in_specs=[_stk_spec(0, DH), _stk_spec(1, DH), _row_spec(DH),
                  _stk_spec(0, DH), _stk_spec(1, DH)],
        out_specs=_row_spec(DC),
        out_shape=jax.ShapeDtypeStruct((N_PAD, DC), jnp.float32),
    )


@functools.cache
def _mesh():
    return plsc.VectorSubcoreMesh(core_axis_name="c", subcore_axis_name="s",
                                  num_cores=NC, num_subcores=NS)


def _deg_kernel_body(dst_hbm, ones_hbm, zeros_hbm, out_hbm,
                     dst_v, ones_v, acc_sh):
    c = lax.axis_index("c")
    s = lax.axis_index("s")
    wid = s * NC + c
    lo = s * ROWS_PER_TILE
    pltpu.sync_copy(dst_hbm.at[wid], dst_v)
    pltpu.sync_copy(ones_hbm, ones_v)
    pltpu.sync_copy(zeros_hbm.at[pl.ds(lo, ROWS_PER_TILE)],
                    acc_sh.at[pl.ds(lo, ROWS_PER_TILE)])
    plsc.subcore_barrier()

    def body(j, carry):
        pltpu.sync_copy(ones_v, acc_sh.at[dst_v.at[j]], add=True)
        return carry

    lax.fori_loop(0, CHUNKS, body, 0)
    plsc.subcore_barrier()
    pltpu.sync_copy(acc_sh.at[pl.ds(lo, ROWS_PER_TILE)],
                    out_hbm.at[c, pl.ds(lo, ROWS_PER_TILE)])


@functools.cache
def _make_deg():
    return pl.kernel(
        _deg_kernel_body,
        out_type=jax.ShapeDtypeStruct((NC, N_PAD, DH), jnp.float32),
        mesh=_mesh(),
        scratch_types=[
            pltpu.VMEM((CHUNKS, CW), jnp.int32),
            pltpu.VMEM((CW, DH), jnp.float32),
            pltpu.VMEM_SHARED((N_PAD, DH), jnp.float32),
        ],
        compiler_params=pltpu.CompilerParams(use_tc_tiling_on_sc=False),
    )


def _agg_kernel_body(h_hbm, src_hbm, dst_hbm, out_hbm,
                     src_v, dst_v, rows_v, acc_sh):
    c = lax.axis_index("c")
    s = lax.axis_index("s")
    wid = s * NC + c
    lo = s * ROWS_PER_TILE
    pltpu.sync_copy(src_hbm.at[wid], src_v)
    pltpu.sync_copy(dst_hbm.at[wid], dst_v)
    pltpu.sync_copy(h_hbm.at[pl.ds(lo, ROWS_PER_TILE)],
                    acc_sh.at[pl.ds(lo, ROWS_PER_TILE)])
    plsc.subcore_barrier()

    def body(j, carry):
        pltpu.sync_copy(h_hbm.at[src_v.at[j]], rows_v)
        pltpu.sync_copy(rows_v, acc_sh.at[dst_v.at[j]], add=True)
        return carry

    lax.fori_loop(0, CHUNKS, body, 0)
    plsc.subcore_barrier()
    pltpu.sync_copy(acc_sh.at[pl.ds(lo, ROWS_PER_TILE)],
                    out_hbm.at[c, pl.ds(lo, ROWS_PER_TILE)])


@functools.cache
def _make_agg():
    return pl.kernel(
        _agg_kernel_body,
        out_type=jax.ShapeDtypeStruct((NC, N_PAD, DH), jnp.float32),
        mesh=_mesh(),
        scratch_types=[
            pltpu.VMEM((CHUNKS, CW), jnp.int32),
            pltpu.VMEM((CHUNKS, CW), jnp.int32),
            pltpu.VMEM((CW, DH), jnp.float32),
            pltpu.VMEM_SHARED((N_PAD, DH), jnp.float32),
        ],
        compiler_params=pltpu.CompilerParams(use_tc_tiling_on_sc=False),
    )


def kernel(x, edge_index, W0, S0, W1, S1, W2, S2):
    src = edge_index[0].astype(jnp.int32).reshape(NW, E_PER_TILE)
    dst = edge_index[1].astype(jnp.int32).reshape(NW, E_PER_TILE)
    pad_n = E_TILE_PAD - E_PER_TILE
    pad_idx = N + (jnp.arange(pad_n, dtype=jnp.int32) % (N_PAD - N))
    src = jnp.concatenate(
        [src, jnp.broadcast_to(pad_idx, (NW, pad_n))],
        axis=1).reshape(NW, CHUNKS, CW)
    dst = jnp.concatenate(
        [dst, jnp.broadcast_to(pad_idx, (NW, pad_n))],
        axis=1).reshape(NW, CHUNKS, CW)

    x_pad = jnp.pad(x, ((0, N_PAD - N), (0, 0)))
    zeros128 = jnp.zeros((N_PAD, DH), jnp.float32)
    ones128 = jnp.ones((CW, DH), jnp.float32)

    Wm0, Wm1, Wm2p = _prep_call(S0, S1, S2, W0, W1, W2)
    dg = _make_deg()(dst, ones128, zeros128)

    h = _make_layer0()(x_pad, Wm0, dg, dg)
    a = _make_agg()(h, src, dst)
    h = _make_layer()(a, a, h, Wm1, dg, dg)
    a = _make_agg()(h, src, dst)
    h = _make_layer()(a, a, h, Wm2p, dg, dg)
    a = _make_agg()(h, src, dst)
    out = _make_final()(a, a, h, dg, dg)
    return out[:N]
